# baseline (speedup 1.0000x reference)
import jax
import jax.numpy as jnp
from jax.experimental import pallas as pl
from jax.experimental.pallas import tpu as pltpu

N_NODES = 50000
K_NB = 31
D = 50
D_ENT = 128
BLK = 400

_BF = jnp.bfloat16
_F32 = jnp.float32


def _body(q_ref, ent_ref, para_ref, score_ref, Wt_ref, bentt_ref, Eq_ref,
          BW_ref, Ee_ref, RW_ref, E31_ref, bbilW_ref, gbias_ref, Wrank_ref,
          brank_ref, out_ref):
    et = jnp.dot(ent_ref[...].astype(_BF), Wt_ref[...],
                 preferred_element_type=_F32) + bentt_ref[...]
    qe = jnp.dot(q_ref[...].astype(_BF), Eq_ref[...],
                 preferred_element_type=_F32)
    u = (qe * et).astype(_BF)
    g2 = jnp.dot(u, BW_ref[...], preferred_element_type=_F32)

    sb = score_ref[...].astype(_BF)
    se = jnp.dot(sb, Ee_ref[...], preferred_element_type=_F32)
    prod = (para_ref[...] * se).astype(_BF)
    h1 = jnp.dot(prod, RW_ref[...], preferred_element_type=_F32)
    s31 = jnp.dot(sb, E31_ref[...], preferred_element_type=_F32)

    pre = h1 + s31 * (g2 + bbilW_ref[...]) + gbias_ref[...]
    on = jnp.where(pre > 0, pre, jnp.exp(jnp.minimum(pre, 0.0)) - 1.0)
    out_ref[...] = jnp.dot(on.astype(_BF), Wrank_ref[...],
                           preferred_element_type=_F32) + brank_ref[...]


@jax.jit
def kernel(query_emb, entity_emb, neighbors_para, neighbors_score, W_ent,
           b_ent, B_bil, b_bil, W_grn, grn_bias, W_rank, b_rank):
    KD = K_NB * D
    DD = D * D
    Wt = jnp.tile(W_ent.T, (1, D)).astype(_BF)
    bentt = jnp.tile(b_ent, D)[None, :]
    Eq = jnp.kron(jnp.eye(D, dtype=_F32), jnp.ones((1, D), _F32)).astype(_BF)
    B_flat = jnp.transpose(B_bil, (1, 2, 0)).reshape(DD, D)
    BW = jnp.dot(B_flat, W_grn.T).astype(_BF)
    Ee = jnp.pad(jnp.kron(jnp.eye(K_NB, dtype=_F32), jnp.ones((1, D), _F32)),
                 ((0, 1), (0, 0))).astype(_BF)
    RW = jnp.tile(W_grn.T, (K_NB, 1)).astype(_BF)
    E31 = jnp.zeros((K_NB + 1, D), _F32).at[K_NB, :].set(1.0).astype(_BF)
    bbilW = jnp.dot(b_bil, W_grn.T)[None, :]
    para2d = neighbors_para.reshape(N_NODES, KD)

    grid = (N_NODES // BLK,)
    c0 = lambda i: (i, 0)
    w0 = lambda i: (0, 0)
    out = pl.pallas_call(
        _body,
        grid=grid,
        in_specs=[
            pl.BlockSpec((BLK, D), c0),
            pl.BlockSpec((BLK, D_ENT), c0),
            pl.BlockSpec((BLK, KD), c0),
            pl.BlockSpec((BLK, K_NB + 1), c0),
            pl.BlockSpec((D_ENT, DD), w0),
            pl.BlockSpec((1, DD), w0),
            pl.BlockSpec((D, DD), w0),
            pl.BlockSpec((DD, D), w0),
            pl.BlockSpec((K_NB + 1, KD), w0),
            pl.BlockSpec((KD, D), w0),
            pl.BlockSpec((K_NB + 1, D), w0),
            pl.BlockSpec((1, D), w0),
            pl.BlockSpec((1, D), w0),
            pl.BlockSpec((D, 1), w0),
            pl.BlockSpec((1, 1), w0),
        ],
        out_specs=pl.BlockSpec((BLK, 1), c0),
        out_shape=jax.ShapeDtypeStruct((N_NODES, 1), _F32),
        compiler_params=pltpu.CompilerParams(
            dimension_semantics=("parallel",)),
    )(query_emb, entity_emb, para2d, neighbors_score,
      Wt, bentt, Eq, BW, Ee, RW, E31, bbilW, grn_bias[None, :],
      W_rank.T.astype(_BF), b_rank[None, :])
    return out


# speedup vs baseline: 7.4789x; 7.4789x over previous
import jax
import jax.numpy as jnp
from jax.experimental import pallas as pl
from jax.experimental.pallas import tpu as pltpu

N_NODES = 50000
K_NB = 31
D = 50
D_ENT = 128
KP = 56
BLK = 512

_BF = jnp.bfloat16
_F32 = jnp.float32


def _body(qT_ref, ent_ref, paraT_ref, scoreT_ref, B2T_ref, Went_ref,
          bent_ref, bbil_ref, Wgrn_ref, gbias_ref, Wrank_ref, brank_ref,
          out_ref):
    qb = qT_ref[...].astype(_BF)
    tT = jnp.dot(B2T_ref[...], qb, preferred_element_type=_F32)
    eb = ent_ref[...].astype(_BF)
    entT = jax.lax.dot_general(Went_ref[...], eb, (((1,), (1,)), ((), ())),
                               preferred_element_type=_F32)
    entT = entT + bent_ref[...]
    acc = entT[0:1, :] * tT[0:KP, :]
    for j in range(1, D):
        acc = acc + entT[j:j + 1, :] * tT[j * KP:(j + 1) * KP, :]
    nodeT = acc[:D, :] + bbil_ref[...]

    score = scoreT_ref[...]
    w = score[K_NB:K_NB + 1, :] * nodeT
    for k in range(K_NB):
        w = w + score[k:k + 1, :] * paraT_ref[:, k, :]
    on = jnp.dot(Wgrn_ref[...], w.astype(_BF),
                 preferred_element_type=_F32) + gbias_ref[...]
    on = jnp.where(on > 0, on, jnp.exp(jnp.minimum(on, 0.0)) - 1.0)
    out_ref[...] = jnp.dot(Wrank_ref[...], on.astype(_BF),
                           preferred_element_type=_F32) + brank_ref[...]


@jax.jit
def kernel(query_emb, entity_emb, neighbors_para, neighbors_score, W_ent,
           b_ent, B_bil, b_bil, W_grn, grn_bias, W_rank, b_rank):
    qT = query_emb.T
    scoreT = neighbors_score.T
    paraT = jnp.transpose(neighbors_para, (2, 1, 0))
    B2T = jnp.transpose(B_bil, (2, 0, 1))
    B2T = jnp.pad(B2T, ((0, 0), (0, KP - D), (0, 0))).reshape(D * KP, D)

    grid = (pl.cdiv(N_NODES, BLK),)
    c0 = lambda i: (0, i)
    w0 = lambda i: (0, 0)
    outT = pl.pallas_call(
        _body,
        grid=grid,
        in_specs=[
            pl.BlockSpec((D, BLK), c0),
            pl.BlockSpec((BLK, D_ENT), lambda i: (i, 0)),
            pl.BlockSpec((D, K_NB, BLK), lambda i: (0, 0, i)),
            pl.BlockSpec((K_NB + 1, BLK), c0),
            pl.BlockSpec((D * KP, D), w0),
            pl.BlockSpec((D, D_ENT), w0),
            pl.BlockSpec((D, 1), w0),
            pl.BlockSpec((D, 1), w0),
            pl.BlockSpec((D, D), w0),
            pl.BlockSpec((D, 1), w0),
            pl.BlockSpec((1, D), w0),
            pl.BlockSpec((1, 1), w0),
        ],
        out_specs=pl.BlockSpec((1, BLK), c0),
        out_shape=jax.ShapeDtypeStruct((1, N_NODES), _F32),
        compiler_params=pltpu.CompilerParams(
            dimension_semantics=("arbitrary",)),
    )(qT, entity_emb, paraT, scoreT,
      B2T.astype(_BF), W_ent.astype(_BF), b_ent[:, None], b_bil[:, None],
      W_grn.astype(_BF), grn_bias[:, None], W_rank.astype(_BF),
      b_rank[:, None])
    return outT.T


# speedup vs baseline: 9.1325x; 1.2211x over previous
import jax
import jax.numpy as jnp
from jax.experimental import pallas as pl
from jax.experimental.pallas import tpu as pltpu

N_NODES = 50000
K_NB = 31
D = 50
D_ENT = 128
KP = 64
BLK = 1024

_BF = jnp.bfloat16
_F32 = jnp.float32


def _body(qT_ref, ent_ref, paraT_ref, scoreT_ref, B2T_ref, Went_ref,
          bent_ref, WgrnA_ref, Wrank_ref, brank_ref, out_ref):
    qb = qT_ref[...].astype(_BF)
    tT = jnp.dot(B2T_ref[...], qb,
                 preferred_element_type=_F32).astype(_BF)
    eb = ent_ref[...].astype(_BF)
    entT = jax.lax.dot_general(Went_ref[...], eb, (((1,), (1,)), ((), ())),
                               preferred_element_type=_F32)
    entT = (entT + bent_ref[...]).astype(_BF)
    acc = (entT[0:1, :] * tT[0:KP, :]).astype(_F32)
    for j in range(1, D):
        acc = acc + entT[j:j + 1, :] * tT[j * KP:(j + 1) * KP, :]
    nodeT = acc[:D, :]

    score = scoreT_ref[...]
    wpara = jnp.sum(paraT_ref[...] * score[None, :K_NB, :], axis=1)
    s31 = score[K_NB:K_NB + 1, :]
    w = wpara + s31 * nodeT
    w_aug = jnp.concatenate(
        [w, s31, jnp.ones((1, w.shape[1]), _F32)], axis=0)
    on = jnp.dot(WgrnA_ref[...], w_aug.astype(_BF),
                 preferred_element_type=_F32)
    on = jnp.where(on > 0, on, jnp.exp(jnp.minimum(on, 0.0)) - 1.0)
    out_ref[...] = jnp.dot(Wrank_ref[...], on.astype(_BF),
                           preferred_element_type=_F32) + brank_ref[...]


@jax.jit
def kernel(query_emb, entity_emb, neighbors_para, neighbors_score, W_ent,
           b_ent, B_bil, b_bil, W_grn, grn_bias, W_rank, b_rank):
    qT = query_emb.T
    scoreT = neighbors_score.T
    paraT = jnp.transpose(neighbors_para, (2, 1, 0))
    B2T = jnp.transpose(B_bil, (2, 0, 1))
    B2T = jnp.pad(B2T, ((0, 0), (0, KP - D), (0, 0))).reshape(D * KP, D)
    WgrnA = jnp.concatenate(
        [W_grn, (W_grn @ b_bil)[:, None], grn_bias[:, None]], axis=1)

    grid = (pl.cdiv(N_NODES, BLK),)
    c0 = lambda i: (0, i)
    w0 = lambda i: (0, 0)
    outT = pl.pallas_call(
        _body,
        grid=grid,
        in_specs=[
            pl.BlockSpec((D, BLK), c0),
            pl.BlockSpec((BLK, D_ENT), lambda i: (i, 0)),
            pl.BlockSpec((D, K_NB, BLK), lambda i: (0, 0, i)),
            pl.BlockSpec((K_NB + 1, BLK), c0),
            pl.BlockSpec((D * KP, D), w0),
            pl.BlockSpec((D, D_ENT), w0),
            pl.BlockSpec((D, 1), w0),
            pl.BlockSpec((D, D + 2), w0),
            pl.BlockSpec((1, D), w0),
            pl.BlockSpec((1, 1), w0),
        ],
        out_specs=pl.BlockSpec((1, BLK), c0),
        out_shape=jax.ShapeDtypeStruct((1, N_NODES), _F32),
        compiler_params=pltpu.CompilerParams(
            dimension_semantics=("arbitrary",)),
    )(qT, entity_emb, paraT, scoreT,
      B2T.astype(_BF), W_ent.astype(_BF), b_ent[:, None],
      WgrnA.astype(_BF), W_rank.astype(_BF), b_rank[:, None])
    return outT.T


# speedup vs baseline: 9.8702x; 1.0808x over previous
import jax
import jax.numpy as jnp
from jax.experimental import pallas as pl
from jax.experimental.pallas import tpu as pltpu

N_NODES = 50000
K_NB = 31
D = 50
D_ENT = 128
KP = 56
BLK = 1024

_BF = jnp.bfloat16
_F32 = jnp.float32


_KC = 16
_DC = 10


def _body(qT_ref, ent_ref, paraT_ref, scoreT_ref, B2T_ref, Went_ref,
          bent_ref, WgrnA_ref, Waux_ref, Wrank_ref, brank_ref, out_ref):
    qb = qT_ref[...].astype(_BF)
    tT = jnp.dot(B2T_ref[...], qb, preferred_element_type=_F32)
    eb = ent_ref[...].astype(_BF)
    entT = jax.lax.dot_general(Went_ref[...], eb, (((1,), (1,)), ((), ())),
                               preferred_element_type=_F32)
    entT = entT + bent_ref[...]
    nodeT = entT[0:1, :] * tT[0:D, :]
    for j in range(1, D):
        nodeT = nodeT + entT[j:j + 1, :] * tT[j * KP:j * KP + D, :]

    score = scoreT_ref[...]
    s31 = score[K_NB:K_NB + 1, :]
    wpara = jnp.sum(paraT_ref[...] * score[None, :K_NB, :], axis=1)
    w = wpara + s31 * nodeT
    aux = jnp.concatenate([s31, jnp.ones((1, w.shape[1]), _F32)], axis=0)
    on = (jnp.dot(WgrnA_ref[...], w.astype(_BF), preferred_element_type=_F32)
          + jnp.dot(Waux_ref[...], aux.astype(_BF),
                    preferred_element_type=_F32))
    on = jnp.where(on > 0, on, jnp.exp(jnp.minimum(on, 0.0)) - 1.0)
    out_ref[...] = jnp.dot(Wrank_ref[...], on.astype(_BF),
                           preferred_element_type=_F32) + brank_ref[...]


@jax.jit
def kernel(query_emb, entity_emb, neighbors_para, neighbors_score, W_ent,
           b_ent, B_bil, b_bil, W_grn, grn_bias, W_rank, b_rank):
    qT = query_emb.T
    scoreT = neighbors_score.T
    paraT = jnp.transpose(neighbors_para, (2, 1, 0))
    B2T = jnp.transpose(B_bil, (2, 0, 1))
    B2T = jnp.pad(B2T, ((0, 0), (0, KP - D), (0, 0))).reshape(D * KP, D)
    Waux = jnp.concatenate([(W_grn @ b_bil)[:, None], grn_bias[:, None]],
                           axis=1)

    grid = (pl.cdiv(N_NODES, BLK),)
    c0 = lambda i: (0, i)
    w0 = lambda i: (0, 0)
    outT = pl.pallas_call(
        _body,
        grid=grid,
        in_specs=[
            pl.BlockSpec((D, BLK), c0),
            pl.BlockSpec((BLK, D_ENT), lambda i: (i, 0)),
            pl.BlockSpec((D, K_NB, BLK), lambda i: (0, 0, i)),
            pl.BlockSpec((K_NB + 1, BLK), c0),
            pl.BlockSpec((D * KP, D), w0),
            pl.BlockSpec((D, D_ENT), w0),
            pl.BlockSpec((D, 1), w0),
            pl.BlockSpec((D, D), w0),
            pl.BlockSpec((D, 2), w0),
            pl.BlockSpec((1, D), w0),
            pl.BlockSpec((1, 1), w0),
        ],
        out_specs=pl.BlockSpec((1, BLK), c0),
        out_shape=jax.ShapeDtypeStruct((1, N_NODES), _F32),
        compiler_params=pltpu.CompilerParams(
            dimension_semantics=("arbitrary",)),
    )(qT, entity_emb, paraT, scoreT,
      B2T.astype(_BF), W_ent.astype(_BF), b_ent[:, None],
      W_grn.astype(_BF), Waux.astype(_BF), W_rank.astype(_BF),
      b_rank[:, None])
    return outT.T
